# baseline (speedup 1.0000x reference)
import jax
import jax.numpy as jnp
from jax import lax
from jax.experimental import pallas as pl
from jax.experimental.pallas import tpu as pltpu
from jax.experimental.pallas import tpu_sc as plsc

_MAXP = 2048
_H = 16
_S = 2048
_TBL = 2 * _MAXP - 1
_W = 4096
_GROUPS_PER_W = 128


def _rpb_body(ws_hbm, out_hbm, ws_v, sem):
    cid = lax.axis_index("c")
    sid = lax.axis_index("s")
    wid = sid * 2 + cid
    h = wid // 2
    half = wid % 2

    for r in range(8):
        pltpu.sync_copy(ws_hbm.at[r, h], ws_v.at[r])

    g0 = half * _GROUPS_PER_W

    def fire(g):
        start = 8 * (255 - g)
        for J in range(16):
            pltpu.async_copy(
                ws_v.at[:, pl.ds(start + 128 * J, 128)],
                out_hbm.at[h, g, J],
                sem,
            )

    for p in range(3):
        fire(g0 + p)

    def step(k, carry):
        @pl.when(k < _GROUPS_PER_W - 3)
        def _():
            fire(g0 + k + 3)
        for _J in range(16):
            pltpu.make_async_copy(
                ws_v.at[:, pl.ds(0, 128)], out_hbm.at[h, 0, 0], sem
            ).wait()
        return carry

    lax.fori_loop(0, _GROUPS_PER_W, step, 0)


@jax.jit
def _rpb_sc(ws):
    mesh = plsc.VectorSubcoreMesh(core_axis_name="c", subcore_axis_name="s")
    return pl.kernel(
        _rpb_body,
        out_type=jax.ShapeDtypeStruct((_H, _S // 8, _S // 128, 8, 128),
                                      jnp.float32),
        mesh=mesh,
        scratch_types=[
            pltpu.VMEM((8, _W), jnp.float32),
            pltpu.SemaphoreType.DMA,
        ],
        compiler_params=pltpu.CompilerParams(use_tc_tiling_on_sc=False),
    )(ws)


def kernel(rel_pos_bias, seq_len):
    del seq_len
    cols = rel_pos_bias.T
    colspad = jnp.pad(cols, ((0, 0), (0, _W + 7 - _TBL)))
    ws = jnp.stack([colspad[:, 7 - r:7 - r + _W] for r in range(8)], axis=0)
    out5 = _rpb_sc(ws)
    return out5.transpose(0, 1, 3, 2, 4).reshape(_H, _S, _S)


# speedup vs baseline: 1.0069x; 1.0069x over previous
import jax
import jax.numpy as jnp
from jax import lax
from jax.experimental import pallas as pl
from jax.experimental.pallas import tpu as pltpu
from jax.experimental.pallas import tpu_sc as plsc

_MAXP = 2048
_H = 16
_S = 2048
_TBL = 2 * _MAXP - 1
_W = 4096
_GROUPS_PER_W = 128


def _rpb_body(ws_hbm, out_hbm, ws_v, band_v, sem):
    cid = lax.axis_index("c")
    sid = lax.axis_index("s")
    wid = sid * 2 + cid
    h = wid // 2
    half = wid % 2

    pltpu.sync_copy(ws_hbm.at[h], ws_v)

    g0 = half * _GROUPS_PER_W

    def fire(g):
        start = 8 * (255 - g)
        for J in range(16):
            pltpu.async_copy(
                ws_v.at[:, pl.ds(start + 128 * J, 128)],
                out_hbm.at[h, g, J],
                sem,
            )

    for p in range(3):
        fire(g0 + p)

    def step(k, carry):
        @pl.when(k < _GROUPS_PER_W - 3)
        def _():
            fire(g0 + k + 3)
        pltpu.make_async_copy(band_v, out_hbm.at[h, 0], sem).wait()
        return carry

    lax.fori_loop(0, _GROUPS_PER_W, step, 0)


@jax.jit
def _rpb_sc(ws):
    mesh = plsc.VectorSubcoreMesh(core_axis_name="c", subcore_axis_name="s")
    return pl.kernel(
        _rpb_body,
        out_type=jax.ShapeDtypeStruct((_H, _S // 8, _S // 128, 8, 128),
                                      jnp.float32),
        mesh=mesh,
        scratch_types=[
            pltpu.VMEM((8, _W), jnp.float32),
            pltpu.VMEM((16, 8, 128), jnp.float32),
            pltpu.SemaphoreType.DMA,
        ],
        compiler_params=pltpu.CompilerParams(use_tc_tiling_on_sc=False),
    )(ws)


def kernel(rel_pos_bias, seq_len):
    del seq_len
    cols = rel_pos_bias.T
    colspad = jnp.pad(cols, ((0, 0), (0, _W + 7 - _TBL)))
    ws = jnp.stack([colspad[:, 7 - r:7 - r + _W] for r in range(8)], axis=1)
    out5 = _rpb_sc(ws)
    return out5.transpose(0, 1, 3, 2, 4).reshape(_H, _S, _S)
